# baseline (speedup 1.0000x reference)
import functools
import math

import jax
import jax.numpy as jnp
from jax import lax
from jax.experimental import pallas as pl
from jax.experimental.pallas import tpu as pltpu
from jax.experimental.pallas import tpu_sc as plsc


def _table_body(feat_ref, w_ref, b_ref, wq_ref, f_ref, q_ref):
    f = (
        jnp.dot(feat_ref[:], w_ref[:], preferred_element_type=jnp.float32)
        + b_ref[:]
    )
    f_ref[:] = f
    q_ref[:] = jnp.dot(f, wq_ref[:], preferred_element_type=jnp.float32)


def _compute_table(feature2, w_fc1, b_fc1_row, w_q):
    n = feature2.shape[0]
    d_model = w_fc1.shape[1]
    return pl.pallas_call(
        _table_body,
        out_shape=(
            jax.ShapeDtypeStruct((n, d_model), jnp.float32),
            jax.ShapeDtypeStruct((n, d_model), jnp.float32),
        ),
    )(feature2, w_fc1, b_fc1_row, w_q)


def _sc_gather(idx_flat, table):
    nk = idx_flat.shape[0]
    d = table.shape[1]
    info = plsc.get_sparse_core_info()
    nw = info.num_cores * info.num_subcores
    per_w = nk // nw
    assert per_w * nw == nk and per_w % 8 == 0
    chunk = 400
    assert per_w % chunk == 0
    n_chunks = per_w // chunk
    mesh = plsc.VectorSubcoreMesh(core_axis_name="c", subcore_axis_name="s")

    @functools.partial(
        pl.kernel,
        out_type=jax.ShapeDtypeStruct((nk, d), jnp.float32),
        mesh=mesh,
        scratch_types=[
            pltpu.VMEM((chunk,), jnp.int32),
            pltpu.VMEM((chunk, d), jnp.float32),
            pltpu.SemaphoreType.DMA,
        ],
    )
    def gather_kernel(idx_hbm, table_hbm, out_hbm, idx_v, rows_v, sem):
        wid = lax.axis_index("s") * info.num_cores + lax.axis_index("c")
        base = wid * per_w

        def body(i, carry):
            off = base + i * chunk
            pltpu.sync_copy(idx_hbm.at[pl.ds(off, chunk)], idx_v)
            pltpu.async_copy(table_hbm.at[idx_v], rows_v, sem).wait()
            pltpu.sync_copy(rows_v, out_hbm.at[pl.ds(off, chunk)])
            return carry

        lax.fori_loop(0, n_chunks, body, 0)

    return gather_kernel(idx_flat, table)


def _attn_body(rel8_ref, knnf_ref, q_ref, feat_ref,
               wd18_ref, bd1_ref, wd2_ref, bd2_ref,
               wkv_ref, wg1_ref, bg1_ref, wg2_ref, bg2_ref,
               wfc2_ref, bfc2_ref, wsc_ref, bsc_ref, out_ref, *, nb, kk, dm):
    nbk = nb * kk
    f32 = jnp.float32

    a2 = jnp.maximum(
        jnp.dot(rel8_ref[:], wd18_ref[:], preferred_element_type=f32)
        + bd1_ref[:], 0.0)
    pos = jnp.dot(a2, wd2_ref[:], preferred_element_type=f32) + bd2_ref[:]

    kv = jnp.dot(knnf_ref[:], wkv_ref[:], preferred_element_type=f32)
    k_ = kv[:, :dm]
    v = kv[:, dm:]

    q_full = jnp.broadcast_to(
        q_ref[:][:, None, :], (nb, kk, dm)).reshape(nbk, dm)
    pre = q_full - k_ + pos
    h = jnp.dot(
        jnp.maximum(jnp.dot(pre, wg1_ref[:], preferred_element_type=f32)
                    + bg1_ref[:], 0.0),
        wg2_ref[:], preferred_element_type=f32,
    ) + bg2_ref[:]
    h = h * (1.0 / math.sqrt(dm))

    h3 = h.reshape(nb, kk, dm)
    m = jnp.max(h3, axis=1, keepdims=True)
    e = jnp.exp(h3 - m)
    s = jnp.sum(e, axis=1, keepdims=True)
    attn3 = e / s

    w3 = attn3 * (v + pos).reshape(nb, kk, dm)
    feat_out = jnp.sum(w3, axis=1)

    out_ref[:] = (
        jnp.dot(feat_out, wfc2_ref[:], preferred_element_type=f32)
        + bfc2_ref[:]
        + jnp.dot(feat_ref[:], wsc_ref[:], preferred_element_type=f32)
        + bsc_ref[:]
    )


def _attn_call(rel8, knnf, q, feature2, wd18, bd1, wd2, bd2, wkv,
               wg1, bg1, wg2, bg2, wfc2, bfc2, wsc, bsc, nb, kk):
    n = q.shape[0]
    dm = wd2.shape[0]
    d_out = wfc2.shape[1]
    grid = n // nb
    assert grid * nb == n

    def blk(i):
        return (i, 0)

    def full(i):
        return (0, 0)

    row_spec = pl.BlockSpec((nb, dm), blk)
    body = functools.partial(_attn_body, nb=nb, kk=kk, dm=dm)

    def w_spec(a):
        return pl.BlockSpec(a.shape, full)

    return pl.pallas_call(
        body,
        grid=(grid,),
        in_specs=[
            pl.BlockSpec((nb * kk, rel8.shape[1]), blk),
            pl.BlockSpec((nb * kk, dm), blk),
            row_spec, pl.BlockSpec((nb, feature2.shape[1]), blk),
            w_spec(wd18), w_spec(bd1), w_spec(wd2), w_spec(bd2),
            w_spec(wkv), w_spec(wg1), w_spec(bg1), w_spec(wg2), w_spec(bg2),
            w_spec(wfc2), w_spec(bfc2), w_spec(wsc), w_spec(bsc),
        ],
        out_specs=pl.BlockSpec((nb, d_out), blk),
        out_shape=jax.ShapeDtypeStruct((n, d_out), jnp.float32),
    )(rel8, knnf, q, feature2, wd18, bd1, wd2, bd2, wkv,
      wg1, bg1, wg2, bg2, wfc2, bfc2, wsc, bsc)


def kernel(xyz, feature, relative_knn_xyz, knn_idx, W_d1, b_d1, W_d2, b_d2,
           W_fc1, b_fc1, W_q, W_k, W_v, W_g1, b_g1, W_g2, b_g2,
           W_fc2, b_fc2, W_sc, b_sc):
    n, kk = knn_idx.shape[1], knn_idx.shape[2]
    feature2 = feature[0]
    rel8 = jnp.pad(relative_knn_xyz[0].reshape(n * kk, 3),
                   ((0, 0), (0, 5)))
    idx_flat = knn_idx[0].reshape(-1)
    wd18 = jnp.pad(W_d1, ((0, 5), (0, 0)))
    wkv = jnp.concatenate([W_k, W_v], axis=1)

    f, q = _compute_table(feature2, W_fc1, b_fc1[None, :], W_q)
    knnf = _sc_gather(idx_flat, f)

    feat = _attn_call(
        rel8, knnf, q, feature2,
        wd18, b_d1[None, :], W_d2, b_d2[None, :], wkv,
        W_g1, b_g1[None, :], W_g2, b_g2[None, :],
        W_fc2, b_fc2[None, :], W_sc, b_sc[None, :],
        nb=200, kk=kk,
    )
    return (xyz, feat[None], relative_knn_xyz, knn_idx)


# speedup vs baseline: 1.2205x; 1.2205x over previous
import functools
import math

import jax
import jax.numpy as jnp
from jax import lax
from jax.experimental import pallas as pl
from jax.experimental.pallas import tpu as pltpu
from jax.experimental.pallas import tpu_sc as plsc


def _table_body(feat_ref, w_ref, b_ref, wq_ref, f_ref, q_ref):
    f = (
        jnp.dot(feat_ref[:], w_ref[:], preferred_element_type=jnp.float32)
        + b_ref[:]
    )
    f_ref[:] = f
    q_ref[:] = jnp.dot(f, wq_ref[:], preferred_element_type=jnp.float32)


def _compute_table(feature2, w_fc1, b_fc1_row, w_q):
    n = feature2.shape[0]
    d_model = w_fc1.shape[1]
    return pl.pallas_call(
        _table_body,
        out_shape=(
            jax.ShapeDtypeStruct((n, d_model), jnp.float32),
            jax.ShapeDtypeStruct((n, d_model), jnp.float32),
        ),
    )(feature2, w_fc1, b_fc1_row, w_q)


def _sc_gather(idx_flat, table):
    nk = idx_flat.shape[0]
    d = table.shape[1]
    info = plsc.get_sparse_core_info()
    nw = info.num_cores * info.num_subcores
    per_w = nk // nw
    assert per_w * nw == nk and per_w % 8 == 0
    chunk = 400
    assert per_w % chunk == 0
    n_chunks = per_w // chunk
    mesh = plsc.VectorSubcoreMesh(core_axis_name="c", subcore_axis_name="s")

    @functools.partial(
        pl.kernel,
        out_type=jax.ShapeDtypeStruct((nk, d), jnp.float32),
        mesh=mesh,
        scratch_types=[
            pltpu.VMEM((chunk,), jnp.int32),
            pltpu.VMEM((chunk, d), jnp.float32),
            pltpu.SemaphoreType.DMA,
        ],
    )
    def gather_kernel(idx_hbm, table_hbm, out_hbm, idx_v, rows_v, sem):
        wid = lax.axis_index("s") * info.num_cores + lax.axis_index("c")
        base = wid * per_w

        def body(i, carry):
            off = base + i * chunk
            pltpu.sync_copy(idx_hbm.at[pl.ds(off, chunk)], idx_v)
            pltpu.async_copy(table_hbm.at[idx_v], rows_v, sem).wait()
            pltpu.sync_copy(rows_v, out_hbm.at[pl.ds(off, chunk)])
            return carry

        lax.fori_loop(0, n_chunks, body, 0)

    return gather_kernel(idx_flat, table)


def _attn_body(rel3_ref, knnf_ref, q_ref, feat_ref,
               wd1_ref, bd1_ref, wd2_ref, bd2_ref,
               wkv_ref, wg1_ref, bg1_ref, wg2_ref, bg2_ref,
               wfc2_ref, bfc2_ref, wsc_ref, bsc_ref, out_ref, *, nb, kk, dm):
    nbk = nb * kk
    f32 = jnp.float32

    a2 = jnp.maximum(
        jnp.dot(rel3_ref[:], wd1_ref[:], preferred_element_type=f32)
        + bd1_ref[:], 0.0)
    pos = jnp.dot(a2, wd2_ref[:], preferred_element_type=f32) + bd2_ref[:]

    kv = jnp.dot(knnf_ref[:], wkv_ref[:], preferred_element_type=f32)
    k_ = kv[:, :dm]
    v = kv[:, dm:]

    q_full = jnp.broadcast_to(
        q_ref[:][:, None, :], (nb, kk, dm)).reshape(nbk, dm)
    pre = q_full - k_ + pos
    h = jnp.dot(
        jnp.maximum(jnp.dot(pre, wg1_ref[:], preferred_element_type=f32)
                    + bg1_ref[:], 0.0),
        wg2_ref[:], preferred_element_type=f32,
    ) + bg2_ref[:]
    h = h * (1.0 / math.sqrt(dm))

    h3 = h.reshape(nb, kk, dm)
    m = jnp.max(h3, axis=1, keepdims=True)
    e = jnp.exp(h3 - m)
    s = jnp.sum(e, axis=1, keepdims=True)
    attn3 = e / s

    w3 = attn3 * (v + pos).reshape(nb, kk, dm)
    feat_out = jnp.sum(w3, axis=1)

    out_ref[:] = (
        jnp.dot(feat_out, wfc2_ref[:], preferred_element_type=f32)
        + bfc2_ref[:]
        + jnp.dot(feat_ref[:], wsc_ref[:], preferred_element_type=f32)
        + bsc_ref[:]
    )


def _attn_call(rel3, knnf, q, feature2, wd1, bd1, wd2, bd2, wkv,
               wg1, bg1, wg2, bg2, wfc2, bfc2, wsc, bsc, nb, kk):
    n = q.shape[0]
    dm = wd2.shape[0]
    d_out = wfc2.shape[1]
    grid = n // nb
    assert grid * nb == n

    def blk(i):
        return (i, 0)

    def full(i):
        return (0, 0)

    row_spec = pl.BlockSpec((nb, dm), blk)
    body = functools.partial(_attn_body, nb=nb, kk=kk, dm=dm)

    def w_spec(a):
        return pl.BlockSpec(a.shape, full)

    return pl.pallas_call(
        body,
        grid=(grid,),
        in_specs=[
            pl.BlockSpec((nb * kk, rel3.shape[1]), blk),
            pl.BlockSpec((nb * kk, dm), blk),
            row_spec, pl.BlockSpec((nb, feature2.shape[1]), blk),
            w_spec(wd1), w_spec(bd1), w_spec(wd2), w_spec(bd2),
            w_spec(wkv), w_spec(wg1), w_spec(bg1), w_spec(wg2), w_spec(bg2),
            w_spec(wfc2), w_spec(bfc2), w_spec(wsc), w_spec(bsc),
        ],
        out_specs=pl.BlockSpec((nb, d_out), blk),
        out_shape=jax.ShapeDtypeStruct((n, d_out), jnp.float32),
    )(rel3, knnf, q, feature2, wd1, bd1, wd2, bd2, wkv,
      wg1, bg1, wg2, bg2, wfc2, bfc2, wsc, bsc)


def kernel(xyz, feature, relative_knn_xyz, knn_idx, W_d1, b_d1, W_d2, b_d2,
           W_fc1, b_fc1, W_q, W_k, W_v, W_g1, b_g1, W_g2, b_g2,
           W_fc2, b_fc2, W_sc, b_sc):
    n, kk = knn_idx.shape[1], knn_idx.shape[2]
    feature2 = feature[0]
    rel3 = relative_knn_xyz[0].reshape(n * kk, 3)
    idx_flat = knn_idx[0].reshape(-1)
    wkv = jnp.concatenate([W_k, W_v], axis=1)

    f, q = _compute_table(feature2, W_fc1, b_fc1[None, :], W_q)
    knnf = _sc_gather(idx_flat, f)

    feat = _attn_call(
        rel3, knnf, q, feature2,
        W_d1, b_d1[None, :], W_d2, b_d2[None, :], wkv,
        W_g1, b_g1[None, :], W_g2, b_g2[None, :],
        W_fc2, b_fc2[None, :], W_sc, b_sc[None, :],
        nb=200, kk=kk,
    )
    return (xyz, feat[None], relative_knn_xyz, knn_idx)


# speedup vs baseline: 1.2263x; 1.0048x over previous
import functools
import math

import jax
import jax.numpy as jnp
from jax import lax
from jax.experimental import pallas as pl
from jax.experimental.pallas import tpu as pltpu
from jax.experimental.pallas import tpu_sc as plsc


def _table_body(feat_ref, w_ref, b_ref, wq_ref, f_ref, q_ref):
    f = (
        jnp.dot(feat_ref[:], w_ref[:], preferred_element_type=jnp.float32)
        + b_ref[:]
    )
    f_ref[:] = f
    q_ref[:] = jnp.dot(f, wq_ref[:], preferred_element_type=jnp.float32)


def _compute_table(feature2, w_fc1, b_fc1_row, w_q):
    n = feature2.shape[0]
    d_model = w_fc1.shape[1]
    return pl.pallas_call(
        _table_body,
        out_shape=(
            jax.ShapeDtypeStruct((n, d_model), jnp.float32),
            jax.ShapeDtypeStruct((n, d_model), jnp.float32),
        ),
    )(feature2, w_fc1, b_fc1_row, w_q)


_NBUF = 5


def _sc_gather(idx_flat, table):
    nk = idx_flat.shape[0]
    d = table.shape[1]
    dt = table.dtype
    info = plsc.get_sparse_core_info()
    nw = info.num_cores * info.num_subcores
    per_w = nk // nw
    assert per_w * nw == nk and per_w % 8 == 0
    chunk = 80
    assert per_w % (chunk * _NBUF) == 0
    n_outer = per_w // (chunk * _NBUF)
    mesh = plsc.VectorSubcoreMesh(core_axis_name="c", subcore_axis_name="s")

    scratch = [pltpu.VMEM((per_w,), jnp.int32)]
    scratch += [pltpu.VMEM((chunk, d), dt) for _ in range(_NBUF)]
    scratch += [pltpu.SemaphoreType.DMA for _ in range(2 * _NBUF)]

    @functools.partial(
        pl.kernel,
        out_type=jax.ShapeDtypeStruct((nk, d), dt),
        mesh=mesh,
        scratch_types=scratch,
    )
    def gather_kernel(idx_hbm, table_hbm, out_hbm, idx_all, *bufs_sems):
        rows = bufs_sems[:_NBUF]
        gsem = bufs_sems[_NBUF:2 * _NBUF]
        ssem = bufs_sems[2 * _NBUF:]
        wid = lax.axis_index("s") * info.num_cores + lax.axis_index("c")
        base = wid * per_w
        pltpu.sync_copy(idx_hbm.at[pl.ds(base, per_w)], idx_all)

        def outer(j, carry):
            for b in range(_NBUF):
                c = j * _NBUF + b

                @pl.when(j > 0)
                def _drain():
                    pltpu.make_async_copy(
                        rows[b], out_hbm.at[pl.ds(base, chunk)], ssem[b]
                    ).wait()

                pltpu.async_copy(
                    table_hbm.at[idx_all.at[pl.ds(c * chunk, chunk)]],
                    rows[b], gsem[b])
            for b in range(_NBUF):
                c = j * _NBUF + b
                pltpu.make_async_copy(
                    table_hbm.at[pl.ds(0, chunk)], rows[b], gsem[b]).wait()
                pltpu.async_copy(
                    rows[b], out_hbm.at[pl.ds(base + c * chunk, chunk)],
                    ssem[b])
            return carry

        lax.fori_loop(0, n_outer, outer, 0)
        for b in range(_NBUF):
            pltpu.make_async_copy(
                rows[b], out_hbm.at[pl.ds(base, chunk)], ssem[b]).wait()

    return gather_kernel(idx_flat, table)


def _attn_body(rel3_ref, knnf_ref, q_ref, feat_ref,
               wd1_ref, bd1_ref, wd2_ref, bd2_ref,
               wkv_ref, wg1_ref, bg1_ref, wg2_ref, bg2_ref,
               wfc2_ref, bfc2_ref, wsc_ref, bsc_ref, out_ref, *, nb, kk, dm):
    nbk = nb * kk
    f32 = jnp.float32

    a2 = jnp.maximum(
        jnp.dot(rel3_ref[:], wd1_ref[:], preferred_element_type=f32)
        + bd1_ref[:], 0.0)
    pos = jnp.dot(a2, wd2_ref[:], preferred_element_type=f32) + bd2_ref[:]

    kv = jnp.dot(knnf_ref[:], wkv_ref[:], preferred_element_type=f32)
    k_ = kv[:, :dm]
    v = kv[:, dm:]

    q_full = jnp.broadcast_to(
        q_ref[:][:, None, :], (nb, kk, dm)).reshape(nbk, dm)
    pre = q_full - k_ + pos
    h = jnp.dot(
        jnp.maximum(jnp.dot(pre, wg1_ref[:], preferred_element_type=f32)
                    + bg1_ref[:], 0.0),
        wg2_ref[:], preferred_element_type=f32,
    ) + bg2_ref[:]
    h = h * (1.0 / math.sqrt(dm))

    h3 = h.reshape(nb, kk, dm)
    m = jnp.max(h3, axis=1, keepdims=True)
    e = jnp.exp(h3 - m)
    s = jnp.sum(e, axis=1, keepdims=True)
    attn3 = e / s

    w3 = attn3 * (v + pos).reshape(nb, kk, dm)
    feat_out = jnp.sum(w3, axis=1)

    out_ref[:] = (
        jnp.dot(feat_out, wfc2_ref[:], preferred_element_type=f32)
        + bfc2_ref[:]
        + jnp.dot(feat_ref[:], wsc_ref[:], preferred_element_type=f32)
        + bsc_ref[:]
    )


def _attn_call(rel3, knnf, q, feature2, wd1, bd1, wd2, bd2, wkv,
               wg1, bg1, wg2, bg2, wfc2, bfc2, wsc, bsc, nb, kk):
    n = q.shape[0]
    dm = wd2.shape[0]
    d_out = wfc2.shape[1]
    grid = n // nb
    assert grid * nb == n

    def blk(i):
        return (i, 0)

    def full(i):
        return (0, 0)

    row_spec = pl.BlockSpec((nb, dm), blk)
    body = functools.partial(_attn_body, nb=nb, kk=kk, dm=dm)

    def w_spec(a):
        return pl.BlockSpec(a.shape, full)

    return pl.pallas_call(
        body,
        grid=(grid,),
        in_specs=[
            pl.BlockSpec((nb * kk, rel3.shape[1]), blk),
            pl.BlockSpec((nb * kk, dm), blk),
            row_spec, pl.BlockSpec((nb, feature2.shape[1]), blk),
            w_spec(wd1), w_spec(bd1), w_spec(wd2), w_spec(bd2),
            w_spec(wkv), w_spec(wg1), w_spec(bg1), w_spec(wg2), w_spec(bg2),
            w_spec(wfc2), w_spec(bfc2), w_spec(wsc), w_spec(bsc),
        ],
        out_specs=pl.BlockSpec((nb, d_out), blk),
        out_shape=jax.ShapeDtypeStruct((n, d_out), jnp.float32),
    )(rel3, knnf, q, feature2, wd1, bd1, wd2, bd2, wkv,
      wg1, bg1, wg2, bg2, wfc2, bfc2, wsc, bsc)


def kernel(xyz, feature, relative_knn_xyz, knn_idx, W_d1, b_d1, W_d2, b_d2,
           W_fc1, b_fc1, W_q, W_k, W_v, W_g1, b_g1, W_g2, b_g2,
           W_fc2, b_fc2, W_sc, b_sc):
    n, kk = knn_idx.shape[1], knn_idx.shape[2]
    feature2 = feature[0]
    rel3 = relative_knn_xyz[0].reshape(n * kk, 3)
    idx_flat = knn_idx[0].reshape(-1)
    wkv = jnp.concatenate([W_k, W_v], axis=1)

    f, q = _compute_table(feature2, W_fc1, b_fc1[None, :], W_q)
    knnf = _sc_gather(idx_flat, f)

    feat = _attn_call(
        rel3, knnf, q, feature2,
        W_d1, b_d1[None, :], W_d2, b_d2[None, :], wkv,
        W_g1, b_g1[None, :], W_g2, b_g2[None, :],
        W_fc2, b_fc2[None, :], W_sc, b_sc[None, :],
        nb=200, kk=kk,
    )
    return (xyz, feat[None], relative_knn_xyz, knn_idx)
